# baseline (speedup 1.0000x reference)
import jax
import jax.numpy as jnp
from jax import lax
from jax.experimental import pallas as pl
from jax.experimental.pallas import tpu as pltpu
from jax.experimental.pallas import tpu_sc as plsc

_B = 16384
_D = 16
_NC = 2
_NS = 16
_NW = _NC * _NS
_BPW = _B // _NW
_CH = 128
_NCH = _BPW // _CH
_TCOL = 8192


def _transpose_body(tT_ref, out_ref):
    out_ref[...] = tT_ref[...].T


def _tc_transpose(tT):
    n = tT.shape[1]
    return pl.pallas_call(
        _transpose_body,
        grid=(pl.cdiv(n, _TCOL),),
        in_specs=[pl.BlockSpec((_D, _TCOL), lambda i: (0, i))],
        out_specs=pl.BlockSpec((_TCOL, _D), lambda i: (i, 0)),
        out_shape=jax.ShapeDtypeStruct((n, _D), jnp.float32),
    )(tT)


def _gather_body(tbl, ids2d, out, idx, rows, sem):
    wid = lax.axis_index("s") * _NC + lax.axis_index("c")
    pltpu.sync_copy(ids2d.at[pl.ds(wid * _NCH, _NCH)], idx)
    copies = [
        pltpu.async_copy(tbl.at[idx.at[j]], rows.at[j], sem)
        for j in range(_NCH)
    ]
    for c in copies:
        c.wait()
    pltpu.sync_copy(rows, out.at[pl.ds(wid * _NCH, _NCH)])


def _sc_gather(table, ids2d):
    mesh = plsc.VectorSubcoreMesh(core_axis_name="c", subcore_axis_name="s")
    f = pl.kernel(
        _gather_body,
        out_type=jax.ShapeDtypeStruct((_NW * _NCH, _CH, _D), jnp.float32),
        mesh=mesh,
        scratch_types=[
            pltpu.VMEM((_NCH, _CH), jnp.int32),
            pltpu.VMEM((_NCH, _CH, _D), jnp.float32),
            pltpu.SemaphoreType.DMA,
        ],
        compiler_params=pltpu.CompilerParams(use_tc_tiling_on_sc=False),
    )
    return f(table, ids2d)


def _mlp_body(ue_ref, ie_ref, w1u_ref, w1i_ref, b1_ref, gamma_ref, beta_ref,
              w2t_ref, b2_ref, out_ref):
    h = jnp.dot(ue_ref[...], w1u_ref[...], preferred_element_type=jnp.float32)
    h = h + jnp.dot(ie_ref[...], w1i_ref[...],
                    preferred_element_type=jnp.float32)
    h = h + b1_ref[...]
    h = jnp.maximum(h, 0.0)
    mean = jnp.mean(h, axis=0, keepdims=True)
    c = h - mean
    var = jnp.mean(c * c, axis=0, keepdims=True)
    hn = c * lax.rsqrt(var + 1e-5) * gamma_ref[...] + beta_ref[...]
    logit = jnp.sum(hn * w2t_ref[...], axis=1) + b2_ref[0]
    out_ref[...] = 1.0 / (1.0 + jnp.exp(-logit))


def _tc_mlp(ue, ie, W1, b1, gamma, beta, W2, b2, interpret=False):
    w1u = W1[:_D, :]
    w1i = W1[_D:, :]
    b1r = b1.reshape(1, _D)
    gr = gamma.reshape(1, _D)
    br = beta.reshape(1, _D)
    w2r = W2.reshape(1, _D)
    b2r = b2.reshape(1)
    return pl.pallas_call(
        _mlp_body,
        out_shape=jax.ShapeDtypeStruct((_B,), jnp.float32),
        in_specs=[
            pl.BlockSpec(memory_space=pltpu.VMEM),
            pl.BlockSpec(memory_space=pltpu.VMEM),
            pl.BlockSpec(memory_space=pltpu.VMEM),
            pl.BlockSpec(memory_space=pltpu.VMEM),
            pl.BlockSpec(memory_space=pltpu.VMEM),
            pl.BlockSpec(memory_space=pltpu.VMEM),
            pl.BlockSpec(memory_space=pltpu.VMEM),
            pl.BlockSpec(memory_space=pltpu.VMEM),
            pl.BlockSpec(memory_space=pltpu.SMEM),
        ],
        out_specs=pl.BlockSpec(memory_space=pltpu.VMEM),
        interpret=interpret,
    )(ue, ie, w1u, w1i, b1r, gr, br, w2r, b2r)


def kernel(user_id, item_id, user_table, item_table, W1, b1, gamma, beta,
           W2, b2):
    uid2d = user_id.reshape(_NW * _NCH, _CH)
    iid2d = item_id.reshape(_NW * _NCH, _CH)
    tu = _tc_transpose(user_table.T)
    ue3 = _sc_gather(tu, uid2d)
    ti = _tc_transpose(item_table.T)
    ie3 = _sc_gather(ti, iid2d)
    ue = ue3.reshape(_B, _D)
    ie = ie3.reshape(_B, _D)
    y = _tc_mlp(ue, ie, W1, b1, gamma, beta, W2, b2)
    return y.reshape(_B, 1)
